# baseline (speedup 1.0000x reference)
import functools

import jax
import jax.numpy as jnp
from jax import lax
from jax.experimental import pallas as pl
from jax.experimental.pallas import tpu as pltpu
from jax.experimental.pallas import tpu_sc as plsc

_B, _C, _H, _W = 8, 17, 512, 512
_ROWS = 256
_SUB = 8
_GRID = (_B, _H // _ROWS)
_NBLK = _GRID[0] * _GRID[1]
_NPIX = float(_B * _H * _W)

_NW = 32
_TROWS = _B * _H // _NW
_CHUNK = 32
_BANK = 32
_NTAB = 8


def _tc_body(x_ref, t_ref, sumT_ref, inter_ref, accD_ref, accI_ref):
    b = pl.program_id(0)
    i = pl.program_id(1)
    pid = b * _GRID[1] + i

    @pl.when(pid == 0)
    def _init():
        accD_ref[...] = jnp.zeros_like(accD_ref)
        accI_ref[...] = jnp.zeros_like(accI_ref)

    cls = jax.lax.broadcasted_iota(jnp.int32, (_C, _SUB, _W), 0)
    for k in range(_ROWS // _SUB):
        xk = x_ref[0, :, pl.ds(k * _SUB, _SUB), :]
        tk = t_ref[0, pl.ds(k * _SUB, _SUB), :]
        Tk = jnp.tanh(0.5 * xk)
        mf2 = jnp.where(cls == tk[None], 2.0, 0.0)
        accD_ref[...] += Tk
        accI_ref[...] += jnp.sum(Tk * mf2, axis=0)

    @pl.when(pid == _NBLK - 1)
    def _finish():
        sumT_ref[...] = jnp.sum(accD_ref[...], axis=1).sum(axis=1)[None, :]
        inter_ref[0, 0] = 0.25 * jnp.sum(accI_ref[...]) + (0.5 * _NPIX)


_SC_MESH = plsc.VectorSubcoreMesh(core_axis_name="c", subcore_axis_name="s")


@functools.partial(
    pl.kernel,
    mesh=_SC_MESH,
    out_type=jax.ShapeDtypeStruct((_NW, _NTAB * 16 * _BANK), jnp.float32),
    scratch_types=[
        pltpu.VMEM((_CHUNK, _W), jnp.int32),
        pltpu.VMEM((_NTAB * 16 * _BANK,), jnp.float32),
    ],
    compiler_params=pltpu.CompilerParams(needs_layout_passes=False),
)
def _sc_histogram(t_hbm, out_hbm, t_v, tab_v):
    wid = lax.axis_index("s") * 2 + lax.axis_index("c")
    base = wid * _TROWS
    lane_base = lax.iota(jnp.int32, 16) * _BANK
    ones = jnp.ones((16,), jnp.float32)

    @pl.loop(0, _NTAB * _BANK)
    def _zero(j):
        tab_v[pl.ds(j * 16, 16)] = jnp.zeros((16,), jnp.float32)

    @pl.loop(0, _TROWS // _CHUNK)
    def _chunk(chunk):
        pltpu.sync_copy(
            t_hbm.at[pl.ds(base + chunk * _CHUNK, _CHUNK)], t_v)

        @pl.loop(0, _CHUNK)
        def _row(r):
            @pl.loop(0, _W // 16, unroll=_NTAB)
            def _col(j):
                lab = t_v[r, pl.ds(j * 16, 16)]
                tab_off = (j % _NTAB) * (16 * _BANK)
                plsc.addupdate_scatter(
                    tab_v, [tab_off + lane_base + lab], ones)

    pltpu.sync_copy(tab_v, out_hbm.at[wid])


def kernel(input, target, smooth):
    target2d = target.reshape(_B * _H, _W)
    table = _sc_histogram(target2d)
    sumT, inter = pl.pallas_call(
        _tc_body,
        grid=_GRID,
        in_specs=[
            pl.BlockSpec((1, _C, _ROWS, _W), lambda b, i: (b, 0, i, 0)),
            pl.BlockSpec((1, _ROWS, _W), lambda b, i: (b, i, 0)),
        ],
        out_specs=[
            pl.BlockSpec((1, _C), lambda b, i: (0, 0)),
            pl.BlockSpec(memory_space=pltpu.SMEM),
        ],
        out_shape=[
            jax.ShapeDtypeStruct((1, _C), jnp.float32),
            jax.ShapeDtypeStruct((1, 1), jnp.float32),
        ],
        scratch_shapes=[
            pltpu.VMEM((_C, _SUB, _W), jnp.float32),
            pltpu.VMEM((_SUB, _W), jnp.float32),
        ],
    )(input, target)
    counts = table.reshape(_NW * _NTAB * 16, _BANK).sum(axis=0)[:_C]
    denom = 0.5 * sumT[0] + (0.5 * _NPIX) + counts
    smooth = smooth.astype(jnp.float32)
    return -(2.0 * inter[0, 0] + jnp.sum(smooth / (denom + smooth)))


# speedup vs baseline: 1.4015x; 1.4015x over previous
import jax
import jax.numpy as jnp
from jax.experimental import pallas as pl
from jax.experimental.pallas import tpu as pltpu

_B, _C, _H, _W = 8, 17, 512, 512
_ROWS = 256
_SUB = 8
_PAIR = 4
_GRID = (_B, _H // _ROWS)
_NBLK = _GRID[0] * _GRID[1]
_NPIX = float(_B * _H * _W)


def _dice_body(smooth_ref, x_ref, t_ref, out_ref, accD_ref, accI_ref):
    b = pl.program_id(0)
    i = pl.program_id(1)
    pid = b * _GRID[1] + i

    @pl.when(pid == 0)
    def _init():
        accD_ref[...] = jnp.zeros_like(accD_ref)
        accI_ref[...] = jnp.zeros_like(accI_ref)

    cls = jax.lax.broadcasted_iota(jnp.int32, (_C, _SUB, _W), 0)
    for g in range(_ROWS // (_SUB * _PAIR)):
        dps = []
        ips = []
        for p in range(_PAIR):
            k = g * _PAIR + p
            xk = x_ref[0, :, pl.ds(k * _SUB, _SUB), :]
            tk = t_ref[0, pl.ds(k * _SUB, _SUB), :]
            Tk = jnp.tanh(0.5 * xk)
            mf2 = jnp.where(cls == tk[None], 2.0, 0.0)
            dps.append(Tk + mf2)
            ips.append(jnp.sum(Tk * mf2, axis=0))
        accD_ref[...] += sum(dps)
        accI_ref[...] += sum(ips)

    @pl.when(pid == _NBLK - 1)
    def _finish():
        smooth = smooth_ref[0, 0]
        denom = 0.5 * jnp.sum(accD_ref[...], axis=(1, 2)) + (0.5 * _NPIX)
        inter = 0.25 * jnp.sum(accI_ref[...]) + (0.5 * _NPIX)
        out_ref[0, 0] = -(2.0 * inter + jnp.sum(smooth / (denom + smooth)))


def kernel(input, target, smooth):
    smooth2d = jnp.reshape(smooth, (1, 1)).astype(jnp.float32)
    out = pl.pallas_call(
        _dice_body,
        grid=_GRID,
        in_specs=[
            pl.BlockSpec(memory_space=pltpu.SMEM),
            pl.BlockSpec((1, _C, _ROWS, _W), lambda b, i: (b, 0, i, 0)),
            pl.BlockSpec((1, _ROWS, _W), lambda b, i: (b, i, 0)),
        ],
        out_specs=pl.BlockSpec(memory_space=pltpu.SMEM),
        out_shape=jax.ShapeDtypeStruct((1, 1), jnp.float32),
        scratch_shapes=[
            pltpu.VMEM((_C, _SUB, _W), jnp.float32),
            pltpu.VMEM((_SUB, _W), jnp.float32),
        ],
    )(smooth2d, input, target)
    return out[0, 0]


# speedup vs baseline: 1.4528x; 1.0366x over previous
import jax
import jax.numpy as jnp
from jax.experimental import pallas as pl
from jax.experimental.pallas import tpu as pltpu

_B, _C, _H, _W = 8, 17, 512, 512
_ROWS = 512
_SUB = 8
_PAIR = 4
_GRID = (_B, _H // _ROWS)
_NBLK = _GRID[0] * _GRID[1]
_NPIX = float(_B * _H * _W)


def _dice_body(smooth_ref, x_ref, t_ref, out_ref, accD_ref, accI_ref):
    b = pl.program_id(0)
    i = pl.program_id(1)
    pid = b * _GRID[1] + i

    @pl.when(pid == 0)
    def _init():
        accD_ref[...] = jnp.zeros_like(accD_ref)
        accI_ref[...] = jnp.zeros_like(accI_ref)

    cls = jax.lax.broadcasted_iota(jnp.int32, (_C, _SUB, _W), 0)
    for g in range(_ROWS // (_SUB * _PAIR)):
        dps = []
        ips = []
        for p in range(_PAIR):
            k = g * _PAIR + p
            xk = x_ref[0, :, pl.ds(k * _SUB, _SUB), :]
            tk = t_ref[0, pl.ds(k * _SUB, _SUB), :]
            Tk = jnp.tanh(0.5 * xk)
            mf2 = jnp.where(cls == tk[None], 2.0, 0.0)
            dps.append(Tk + mf2)
            ips.append(jnp.sum(Tk * mf2, axis=0))
        accD_ref[...] += sum(dps)
        accI_ref[...] += sum(ips)

    @pl.when(pid == _NBLK - 1)
    def _finish():
        smooth = smooth_ref[0, 0]
        denom = 0.5 * jnp.sum(accD_ref[...], axis=(1, 2)) + (0.5 * _NPIX)
        inter = 0.25 * jnp.sum(accI_ref[...]) + (0.5 * _NPIX)
        out_ref[0, 0] = -(2.0 * inter + jnp.sum(smooth / (denom + smooth)))


def kernel(input, target, smooth):
    smooth2d = jnp.reshape(smooth, (1, 1)).astype(jnp.float32)
    out = pl.pallas_call(
        _dice_body,
        grid=_GRID,
        in_specs=[
            pl.BlockSpec(memory_space=pltpu.SMEM),
            pl.BlockSpec((1, _C, _ROWS, _W), lambda b, i: (b, 0, i, 0)),
            pl.BlockSpec((1, _ROWS, _W), lambda b, i: (b, i, 0)),
        ],
        out_specs=pl.BlockSpec(memory_space=pltpu.SMEM),
        out_shape=jax.ShapeDtypeStruct((1, 1), jnp.float32),
        scratch_shapes=[
            pltpu.VMEM((_C, _SUB, _W), jnp.float32),
            pltpu.VMEM((_SUB, _W), jnp.float32),
        ],
    )(smooth2d, input, target)
    return out[0, 0]
